# baseline (speedup 1.0000x reference)
import functools

import jax
import jax.numpy as jnp
from jax import lax
from jax.experimental import pallas as pl
from jax.experimental.pallas import tpu as pltpu
from jax.experimental.pallas import tpu_sc as plsc

_GDN = lax.GatherDimensionNumbers(
    offset_dims=(), collapsed_slice_dims=(0,), start_index_map=(0,))


def _permute(v, idx):
    return lax.gather(v, idx[:, None], _GDN, (1,),
                      mode=lax.GatherScatterMode.PROMISE_IN_BOUNDS)


N_USER = 25000
N_ITEM = 25000
N_NODE = N_USER + N_ITEM
N_EDGE = 800000
D = 64
CD = 32

N_PAD = 50176
E_PAD = 819200
N_WORKER = 32
EPW = E_PAD // N_WORKER
CHUNK = 128
NCHUNK = EPW // CHUNK
NITER = NCHUNK // 4
ROWS_PER_TILE = N_PAD // 16


def _merge_body(ups_ref, ue_ref, wt_ref, wb_ref, b_ref, o_ref):
    acc = jnp.dot(ups_ref[...], wt_ref[...], preferred_element_type=jnp.float32)
    acc += jnp.dot(ue_ref[...], wb_ref[...], preferred_element_type=jnp.float32)
    o_ref[...] = acc + b_ref[...]


def _merge_users(ups, ue, w_merge, b_merge):
    blk = 1000
    grid = N_USER // blk
    return pl.pallas_call(
        _merge_body,
        grid=(grid,),
        in_specs=[
            pl.BlockSpec((blk, D), lambda i: (i, 0)),
            pl.BlockSpec((blk, D), lambda i: (i, 0)),
            pl.BlockSpec((D, D), lambda i: (0, 0)),
            pl.BlockSpec((D, D), lambda i: (0, 0)),
            pl.BlockSpec((1, D), lambda i: (0, 0)),
        ],
        out_specs=pl.BlockSpec((blk, D), lambda i: (i, 0)),
        out_shape=jax.ShapeDtypeStruct((N_USER, D), jnp.float32),
    )(ups, ue, w_merge[:D], w_merge[D:], b_merge[None, :])


def _chan_body(embT_ref, wT_ref, bT_ref, z0_ref, z1_ref):
    embT = embT_ref[...]
    for c, out in ((0, z0_ref), (1, z1_ref)):
        zT = jnp.dot(wT_ref[c], embT, preferred_element_type=jnp.float32)
        zT = zT + bT_ref[c]
        nrm = jnp.sqrt(jnp.sum(zT * zT, axis=0, keepdims=True))
        out[...] = zT / (nrm + 1e-12)


def _channel_proj(embT, w_chT, b_chT):
    blk = 512
    grid = N_PAD // blk
    return pl.pallas_call(
        _chan_body,
        grid=(grid,),
        in_specs=[
            pl.BlockSpec((D, blk), lambda i: (0, i)),
            pl.BlockSpec((2, CD, D), lambda i: (0, 0, 0)),
            pl.BlockSpec((2, CD, 1), lambda i: (0, 0, 0)),
        ],
        out_specs=[
            pl.BlockSpec((CD, blk), lambda i: (0, i)),
            pl.BlockSpec((CD, blk), lambda i: (0, i)),
        ],
        out_shape=[
            jax.ShapeDtypeStruct((CD, N_PAD), jnp.float32),
            jax.ShapeDtypeStruct((CD, N_PAD), jnp.float32),
        ],
    )(embT, w_chT, b_chT)


RITER = 392


def _repack_body(z0T, z1T, embT, z0t, z1t, emb_tab, stg, stge, st32, st64, sem):
    cid = lax.axis_index("c")
    sid = lax.axis_index("s")
    wid = sid * 2 + cid
    base_n = wid * (N_PAD // 32)
    rows_i = lax.iota(jnp.int32, 16)

    for it in range(4):
        n0 = base_n + it * RITER
        for tabT, tab_out in ((z0T, z0t), (z1T, z1t)):
            cps = [pltpu.async_copy(tabT.at[f, pl.ds(n0, RITER)],
                                    stg.at[pl.ds(f * RITER, RITER)], sem)
                   for f in range(CD)]
            for cp in cps:
                cp.wait()

            def _tp(n, carry):
                for q in range(2):
                    idx = (rows_i + q * 16) * RITER + n
                    st32[n, pl.ds(q * 16, 16)] = plsc.load_gather(stg, [idx])
                return carry
            lax.fori_loop(0, RITER, _tp, 0)
            pltpu.sync_copy(st32, tab_out.at[pl.ds(n0, RITER)])

        cps = [pltpu.async_copy(embT.at[f, pl.ds(n0, RITER)],
                                stge.at[pl.ds(f * RITER, RITER)], sem)
               for f in range(D)]
        for cp in cps:
            cp.wait()

        def _tpe(n, carry):
            for q in range(4):
                idx = (rows_i + q * 16) * RITER + n
                st64[n, pl.ds(q * 16, 16)] = plsc.load_gather(stge, [idx])
            return carry
        lax.fori_loop(0, RITER, _tpe, 0)
        pltpu.sync_copy(st64, emb_tab.at[pl.ds(n0, RITER)])


def _sc_repack(z0T, z1T, embT):
    mesh = plsc.VectorSubcoreMesh(core_axis_name="c", subcore_axis_name="s",
                                  num_cores=2, num_subcores=16)
    fn = pl.kernel(
        _repack_body,
        out_type=[
            jax.ShapeDtypeStruct((N_PAD, CD), jnp.float32),
            jax.ShapeDtypeStruct((N_PAD, CD), jnp.float32),
            jax.ShapeDtypeStruct((N_PAD, D), jnp.float32),
        ],
        mesh=mesh,
        compiler_params=pltpu.CompilerParams(needs_layout_passes=False,
                                             use_tc_tiling_on_sc=False),
        scratch_types=[
            pltpu.VMEM((CD * RITER,), jnp.float32),
            pltpu.VMEM((D * RITER,), jnp.float32),
            pltpu.VMEM((RITER, CD), jnp.float32),
            pltpu.VMEM((RITER, D), jnp.float32),
            pltpu.SemaphoreType.DMA,
        ],
    )
    return fn(z0T, z1T, embT)


def _sc_body(z0_hbm, z1_hbm, row_hbm, col_hbm, num_out,
             den00, den01, den10, den11,
             idxr, idxc, zsrc, zdst, wz, w2d, zvec,
             num_sh, den_sh,
             isem0, isem1, gsem0, gsem1, ssem0, ssem1):
    cid = lax.axis_index("c")
    sid = lax.axis_index("s")
    wid = sid * 2 + cid
    tbase = sid * ROWS_PER_TILE
    widbase = wid * NCHUNK
    isems = (isem0, isem1)
    gsems = (gsem0, gsem1)
    ssems = (ssem0, ssem1)

    rows_i = lax.iota(jnp.int32, 16)
    xor_idx = [rows_i ^ d for d in (8, 4, 2, 1)]
    conds = [(rows_i & d) == 0 for d in (8, 4, 2, 1)]

    def _merge(x, y, r):
        a = x + _permute(x, xor_idx[r])
        b = y + _permute(y, xor_idx[r])
        return jnp.where(conds[r], a, b)

    def _fire_idx(k, slot, p):
        pltpu.async_copy(row_hbm.at[widbase + k], idxr.at[slot], isems[p])
        pltpu.async_copy(col_hbm.at[widbase + k], idxc.at[slot], isems[p])

    def _wait_idx(p):
        pltpu.make_async_copy(row_hbm.at[0], idxr.at[0], isems[p]).wait()
        pltpu.make_async_copy(row_hbm.at[0], idxc.at[0], isems[p]).wait()

    def _fire_gather(ztab, islot, dslot, p):
        pltpu.async_copy(ztab.at[idxr.at[islot]], zsrc.at[dslot], gsems[p])
        pltpu.async_copy(ztab.at[idxc.at[islot]], zdst.at[dslot], gsems[p])

    def _wait_gather(p):
        pltpu.make_async_copy(z0_hbm.at[pl.ds(0, CHUNK)], zsrc.at[0],
                              gsems[p]).wait()
        pltpu.make_async_copy(z0_hbm.at[pl.ds(0, CHUNK)], zdst.at[0],
                              gsems[p]).wait()

    def _fire_scatter(islot, dslot, p):
        pltpu.async_copy(wz.at[dslot], num_sh.at[idxr.at[islot]],
                         ssems[p], add=True)
        pltpu.async_copy(w2d.at[dslot], den_sh.at[idxr.at[islot]],
                         ssems[p], add=True)

    def _wait_scatter(p):
        pltpu.make_async_copy(z0_hbm.at[pl.ds(0, CHUNK)], wz.at[0],
                              ssems[p]).wait()
        pltpu.make_async_copy(den00.at[pl.ds(0, CHUNK)], w2d.at[0],
                              ssems[p]).wait()

    def _compute(d):
        def _grp(g, carry2):
            base = g * 16
            vecs = []
            bregs = []
            for e in range(16):
                a0 = zsrc[d, base + e, pl.ds(0, 16)]
                a1 = zsrc[d, base + e, pl.ds(16, 16)]
                b0 = zdst[d, base + e, pl.ds(0, 16)]
                b1 = zdst[d, base + e, pl.ds(16, 16)]
                bregs.append((b0, b1))
                vecs.append(a0 * b0 + a1 * b1)
            for r in range(4):
                half = len(vecs) // 2
                vecs = [_merge(vecs[i], vecs[i + half], r)
                        for i in range(half)]
            dots = vecs[0]
            sv = jnp.where(dots >= 0.0, dots, dots * 0.01)
            wv = jnp.exp(sv)
            w2d[d, pl.ds(g * 16, 16)] = wv
            for e in range(16):
                ws = _permute(wv, jnp.full((16,), e, jnp.int32))
                b0, b1 = bregs[e]
                wz[d, base + e, pl.ds(0, 16)] = b0 * ws
                wz[d, base + e, pl.ds(16, 16)] = b1 * ws
            return carry2
        lax.fori_loop(0, CHUNK // 16, _grp, 0)

    def _zv(i, _):
        zvec[pl.ds(i * 16, 16)] = jnp.zeros((16,), jnp.float32)
        return _
    lax.fori_loop(0, 448 // 16, _zv, 0)

    for ch in range(2):
        ztab = z0_hbm if ch == 0 else z1_hbm

        _fire_idx(0, 0, 0)
        _wait_idx(0)
        _fire_gather(ztab, 0, 0, 0)
        _fire_idx(1, 1, 1)

        def _zw(i, _):
            wz[0, i, pl.ds(0, 16)] = jnp.zeros((16,), jnp.float32)
            wz[0, i, pl.ds(16, 16)] = jnp.zeros((16,), jnp.float32)
            return _
        lax.fori_loop(0, CHUNK, _zw, 0)
        for j in range(24):
            pltpu.sync_copy(wz.at[0],
                            num_sh.at[pl.ds(tbase + j * 128, 128)])
        pltpu.sync_copy(wz.at[0, pl.ds(0, 64)],
                        num_sh.at[pl.ds(tbase + 3072, 64)])
        for j in range(7):
            pltpu.sync_copy(zvec, den_sh.at[pl.ds(tbase + j * 448, 448)])
        plsc.subcore_barrier()

        def _body(i, carry):
            for s in range(4):
                k = i * 4 + s
                p = s % 2
                if s < 2:
                    @pl.when(i > 0)
                    def _w1():
                        _wait_scatter(p)
                else:
                    _wait_scatter(p)
                if s < 2:
                    _fire_idx(k + 2, s + 2, p)
                else:
                    @pl.when(i < NITER - 1)
                    def _f2():
                        _fire_idx(k + 2, s - 2, p)
                if s < 3:
                    _wait_idx(1 - p)
                    _fire_gather(ztab, (s + 1) % 4, 1 - p, 1 - p)
                else:
                    @pl.when(i < NITER - 1)
                    def _f3():
                        _wait_idx(1 - p)
                        _fire_gather(ztab, 0, 1 - p, 1 - p)
                _wait_gather(p)
                _compute(p)
                _fire_scatter(s, p, p)
            return carry
        lax.fori_loop(0, NITER, _body, 0)
        _wait_scatter(0)
        _wait_scatter(1)
        plsc.subcore_barrier()

        pltpu.sync_copy(num_sh.at[pl.ds(tbase, ROWS_PER_TILE)],
                        num_out.at[ch, cid, pl.ds(tbase, ROWS_PER_TILE)])
        den_c0 = (den00, den10)[ch]
        den_c1 = (den01, den11)[ch]

        @pl.when(cid == 0)
        def _flush0():
            pltpu.sync_copy(den_sh.at[pl.ds(tbase, ROWS_PER_TILE)],
                            den_c0.at[pl.ds(tbase, ROWS_PER_TILE)])

        @pl.when(cid == 1)
        def _flush1():
            pltpu.sync_copy(den_sh.at[pl.ds(tbase, ROWS_PER_TILE)],
                            den_c1.at[pl.ds(tbase, ROWS_PER_TILE)])
        plsc.subcore_barrier()


def _sc_aggregate(z0, z1, row2d, col2d):
    mesh = plsc.VectorSubcoreMesh(core_axis_name="c", subcore_axis_name="s",
                                  num_cores=2, num_subcores=16)
    fn = pl.kernel(
        _sc_body,
        out_type=[
            jax.ShapeDtypeStruct((2, 2, N_PAD, CD), jnp.float32),
            jax.ShapeDtypeStruct((N_PAD,), jnp.float32),
            jax.ShapeDtypeStruct((N_PAD,), jnp.float32),
            jax.ShapeDtypeStruct((N_PAD,), jnp.float32),
            jax.ShapeDtypeStruct((N_PAD,), jnp.float32),
        ],
        mesh=mesh,
        compiler_params=pltpu.CompilerParams(needs_layout_passes=False,
                                             use_tc_tiling_on_sc=False),
        scratch_types=[
            pltpu.VMEM((4, CHUNK), jnp.int32),
            pltpu.VMEM((4, CHUNK), jnp.int32),
            pltpu.VMEM((2, CHUNK, CD), jnp.float32),
            pltpu.VMEM((2, CHUNK, CD), jnp.float32),
            pltpu.VMEM((2, CHUNK, CD), jnp.float32),
            pltpu.VMEM((2, CHUNK), jnp.float32),
            pltpu.VMEM((448,), jnp.float32),
            pltpu.VMEM_SHARED((N_PAD, CD), jnp.float32),
            pltpu.VMEM_SHARED((N_PAD,), jnp.float32),
            pltpu.SemaphoreType.DMA,
            pltpu.SemaphoreType.DMA,
            pltpu.SemaphoreType.DMA,
            pltpu.SemaphoreType.DMA,
            pltpu.SemaphoreType.DMA,
            pltpu.SemaphoreType.DMA,
        ],
    )
    return fn(z0, z1, row2d, col2d)


NODES_PER_TILE_F = N_PAD // 32
FCHUNK = 224


def _fin_body(num, d00, d01, d10, d11, emb_tab, out128,
              nb00, nb01, nb10, nb11, db00, db01, db10, db11, eb, ob):
    cid = lax.axis_index("c")
    sid = lax.axis_index("s")
    wid = sid * 2 + cid

    def _chunk(c, carry):
        nb = wid * NODES_PER_TILE_F + c * FCHUNK
        for core, (nref0, nref1) in ((0, (nb00, nb10)), (1, (nb01, nb11))):
            pltpu.sync_copy(num.at[0, core, pl.ds(nb, FCHUNK)], nref0)
            pltpu.sync_copy(num.at[1, core, pl.ds(nb, FCHUNK)], nref1)
        for src, dst in ((d00, db00), (d01, db01), (d10, db10), (d11, db11)):
            pltpu.sync_copy(src.at[pl.ds(nb, FCHUNK)], dst)
        pltpu.sync_copy(emb_tab.at[pl.ds(nb, FCHUNK)], eb)

        def _grp(g, carry2):
            g16 = g * 16
            dv0 = db00[pl.ds(g16, 16)] + db01[pl.ds(g16, 16)]
            rec0 = 1.0 / (dv0 + 1e-16)
            dv1 = db10[pl.ds(g16, 16)] + db11[pl.ds(g16, 16)]
            rec1 = 1.0 / (dv1 + 1e-16)
            for e in range(16):
                n = g16 + e
                r0e = _permute(rec0, jnp.full((16,), e, jnp.int32))
                r1e = _permute(rec1, jnp.full((16,), e, jnp.int32))
                a0 = (nb00[n, pl.ds(0, 16)] + nb01[n, pl.ds(0, 16)]) * r0e
                a1 = (nb00[n, pl.ds(16, 16)] + nb01[n, pl.ds(16, 16)]) * r0e
                b0 = (nb10[n, pl.ds(0, 16)] + nb11[n, pl.ds(0, 16)]) * r1e
                b1 = (nb10[n, pl.ds(16, 16)] + nb11[n, pl.ds(16, 16)]) * r1e
                er = g * 8 + e // 2
                ec = (e % 2) * 64
                ob[er, pl.ds(ec, 16)] = 0.5 * (eb[n, pl.ds(0, 16)] + a0)
                ob[er, pl.ds(ec + 16, 16)] = 0.5 * (eb[n, pl.ds(16, 16)] + a1)
                ob[er, pl.ds(ec + 32, 16)] = 0.5 * (eb[n, pl.ds(32, 16)] + b0)
                ob[er, pl.ds(ec + 48, 16)] = 0.5 * (eb[n, pl.ds(48, 16)] + b1)
            return carry2
        lax.fori_loop(0, FCHUNK // 16, _grp, 0)
        pltpu.sync_copy(ob, out128.at[pl.ds(nb // 2, FCHUNK // 2)])
        return carry
    lax.fori_loop(0, NODES_PER_TILE_F // FCHUNK, _chunk, 0)


def _sc_finalize(num, d00, d01, d10, d11, emb_tab):
    mesh = plsc.VectorSubcoreMesh(core_axis_name="c", subcore_axis_name="s",
                                  num_cores=2, num_subcores=16)
    fn = pl.kernel(
        _fin_body,
        out_type=jax.ShapeDtypeStruct((N_PAD * D // 128, 128), jnp.float32),
        mesh=mesh,
        compiler_params=pltpu.CompilerParams(needs_layout_passes=False,
                                             use_tc_tiling_on_sc=False),
        scratch_types=[
            pltpu.VMEM((FCHUNK, CD), jnp.float32),
            pltpu.VMEM((FCHUNK, CD), jnp.float32),
            pltpu.VMEM((FCHUNK, CD), jnp.float32),
            pltpu.VMEM((FCHUNK, CD), jnp.float32),
            pltpu.VMEM((FCHUNK,), jnp.float32),
            pltpu.VMEM((FCHUNK,), jnp.float32),
            pltpu.VMEM((FCHUNK,), jnp.float32),
            pltpu.VMEM((FCHUNK,), jnp.float32),
            pltpu.VMEM((FCHUNK, D), jnp.float32),
            pltpu.VMEM((FCHUNK // 2, 128), jnp.float32),
        ],
    )
    return fn(num, d00, d01, d10, d11, emb_tab)


def kernel(user_emb, item_emb, user_preference_sample, edge_index,
           W_ch, b_ch, W_merge, b_merge):
    u = _merge_users(user_preference_sample, user_emb, W_merge, b_merge)
    emb = jnp.concatenate([u, item_emb], axis=0)
    embT = jnp.pad(emb, ((0, N_PAD - N_NODE), (0, 0))).T

    z0T, z1T = _channel_proj(embT, W_ch.transpose(0, 2, 1),
                             b_ch.transpose(0, 2, 1))
    z0t, z1t, emb_tab = _sc_repack(z0T, z1T, embT)

    pad = jnp.full((E_PAD - N_EDGE,), N_NODE, dtype=jnp.int32)
    row2d = jnp.concatenate([edge_index[0], pad]).reshape(E_PAD // 128, 128)
    col2d = jnp.concatenate([edge_index[1], pad]).reshape(E_PAD // 128, 128)

    num, d00, d01, d10, d11 = _sc_aggregate(z0t, z1t, row2d, col2d)

    out128 = _sc_finalize(num, d00, d01, d10, d11, emb_tab)
    return out128.reshape(N_PAD, D)[:N_NODE]


# speedup vs baseline: 2.0672x; 2.0672x over previous
import functools

import jax
import jax.numpy as jnp
from jax import lax
from jax.experimental import pallas as pl
from jax.experimental.pallas import tpu as pltpu
from jax.experimental.pallas import tpu_sc as plsc

_GDN = lax.GatherDimensionNumbers(
    offset_dims=(), collapsed_slice_dims=(0,), start_index_map=(0,))


def _permute(v, idx):
    return lax.gather(v, idx[:, None], _GDN, (1,),
                      mode=lax.GatherScatterMode.PROMISE_IN_BOUNDS)


N_USER = 25000
N_ITEM = 25000
N_NODE = N_USER + N_ITEM
N_EDGE = 800000
D = 64
CD = 32

N_PAD = 50176
E_PAD = 819200
N_WORKER = 32
EPW = E_PAD // N_WORKER
CHUNK = 128
NCHUNK = EPW // CHUNK
NITER = NCHUNK // 4
ROWS_PER_TILE = N_PAD // 16


def _merge_body(ups_ref, ue_ref, wt_ref, wb_ref, b_ref, o_ref):
    acc = jnp.dot(ups_ref[...], wt_ref[...], preferred_element_type=jnp.float32)
    acc += jnp.dot(ue_ref[...], wb_ref[...], preferred_element_type=jnp.float32)
    o_ref[...] = acc + b_ref[...]


def _merge_users(ups, ue, w_merge, b_merge):
    blk = 1000
    grid = N_USER // blk
    return pl.pallas_call(
        _merge_body,
        grid=(grid,),
        in_specs=[
            pl.BlockSpec((blk, D), lambda i: (i, 0)),
            pl.BlockSpec((blk, D), lambda i: (i, 0)),
            pl.BlockSpec((D, D), lambda i: (0, 0)),
            pl.BlockSpec((D, D), lambda i: (0, 0)),
            pl.BlockSpec((1, D), lambda i: (0, 0)),
        ],
        out_specs=pl.BlockSpec((blk, D), lambda i: (i, 0)),
        out_shape=jax.ShapeDtypeStruct((N_USER, D), jnp.float32),
    )(ups, ue, w_merge[:D], w_merge[D:], b_merge[None, :])


def _chan_body(embT_ref, wT_ref, bT_ref, z0_ref, z1_ref):
    embT = embT_ref[...]
    for c, out in ((0, z0_ref), (1, z1_ref)):
        zT = jnp.dot(wT_ref[c], embT, preferred_element_type=jnp.float32)
        zT = zT + bT_ref[c]
        nrm = jnp.sqrt(jnp.sum(zT * zT, axis=0, keepdims=True))
        out[...] = zT / (nrm + 1e-12)


def _channel_proj(embT, w_chT, b_chT):
    blk = 512
    grid = N_PAD // blk
    return pl.pallas_call(
        _chan_body,
        grid=(grid,),
        in_specs=[
            pl.BlockSpec((D, blk), lambda i: (0, i)),
            pl.BlockSpec((2, CD, D), lambda i: (0, 0, 0)),
            pl.BlockSpec((2, CD, 1), lambda i: (0, 0, 0)),
        ],
        out_specs=[
            pl.BlockSpec((CD, blk), lambda i: (0, i)),
            pl.BlockSpec((CD, blk), lambda i: (0, i)),
        ],
        out_shape=[
            jax.ShapeDtypeStruct((CD, N_PAD), jnp.float32),
            jax.ShapeDtypeStruct((CD, N_PAD), jnp.float32),
        ],
    )(embT, w_chT, b_chT)


RITER = 392


def _repack_body(z0T, z1T, embT, z0t, z1t, emb_tab, stg, stge, st32, st64, sem):
    cid = lax.axis_index("c")
    sid = lax.axis_index("s")
    wid = sid * 2 + cid
    base_n = wid * (N_PAD // 32)
    rows_i = lax.iota(jnp.int32, 16)

    for it in range(4):
        n0 = base_n + it * RITER
        for tabT, tab_out in ((z0T, z0t), (z1T, z1t)):
            cps = [pltpu.async_copy(tabT.at[f, pl.ds(n0, RITER)],
                                    stg.at[pl.ds(f * RITER, RITER)], sem)
                   for f in range(CD)]
            for cp in cps:
                cp.wait()

            def _tp(n, carry):
                for q in range(2):
                    idx = (rows_i + q * 16) * RITER + n
                    st32[n, pl.ds(q * 16, 16)] = plsc.load_gather(stg, [idx])
                return carry
            lax.fori_loop(0, RITER, _tp, 0)
            pltpu.sync_copy(st32, tab_out.at[pl.ds(n0, RITER)])

        cps = [pltpu.async_copy(embT.at[f, pl.ds(n0, RITER)],
                                stge.at[pl.ds(f * RITER, RITER)], sem)
               for f in range(D)]
        for cp in cps:
            cp.wait()

        def _tpe(n, carry):
            for q in range(4):
                idx = (rows_i + q * 16) * RITER + n
                st64[n, pl.ds(q * 16, 16)] = plsc.load_gather(stge, [idx])
            return carry
        lax.fori_loop(0, RITER, _tpe, 0)
        pltpu.sync_copy(st64, emb_tab.at[pl.ds(n0, RITER)])


def _sc_repack(z0T, z1T, embT):
    mesh = plsc.VectorSubcoreMesh(core_axis_name="c", subcore_axis_name="s",
                                  num_cores=2, num_subcores=16)
    fn = pl.kernel(
        _repack_body,
        out_type=[
            jax.ShapeDtypeStruct((N_PAD, CD), jnp.float32),
            jax.ShapeDtypeStruct((N_PAD, CD), jnp.float32),
            jax.ShapeDtypeStruct((N_PAD, D), jnp.float32),
        ],
        mesh=mesh,
        compiler_params=pltpu.CompilerParams(needs_layout_passes=False,
                                             use_tc_tiling_on_sc=False),
        scratch_types=[
            pltpu.VMEM((CD * RITER,), jnp.float32),
            pltpu.VMEM((D * RITER,), jnp.float32),
            pltpu.VMEM((RITER, CD), jnp.float32),
            pltpu.VMEM((RITER, D), jnp.float32),
            pltpu.SemaphoreType.DMA,
        ],
    )
    return fn(z0T, z1T, embT)


def _sc_body(z0_hbm, z1_hbm, row_hbm, col_hbm, num_out,
             den00, den01, den10, den11,
             idxr, idxc, zsrc, zdst, wz, w2d, zvec,
             num_sh, den_sh,
             isem0, isem1, gsem0, gsem1, ssem0, ssem1):
    cid = lax.axis_index("c")
    sid = lax.axis_index("s")
    wid = sid * 2 + cid
    tbase = sid * ROWS_PER_TILE
    widbase = wid * NCHUNK
    isems = (isem0, isem1)
    gsems = (gsem0, gsem1)
    ssems = (ssem0, ssem1)

    rows_i = lax.iota(jnp.int32, 16)
    xor_idx = [rows_i ^ d for d in (8, 4, 2, 1)]
    conds = [(rows_i & d) == 0 for d in (8, 4, 2, 1)]

    def _merge(x, y, r):
        a = x + _permute(x, xor_idx[r])
        b = y + _permute(y, xor_idx[r])
        return jnp.where(conds[r], a, b)

    def _fire_idx(k, slot, p):
        pltpu.async_copy(row_hbm.at[widbase + k], idxr.at[slot], isems[p])
        pltpu.async_copy(col_hbm.at[widbase + k], idxc.at[slot], isems[p])

    def _wait_idx(p):
        pltpu.make_async_copy(row_hbm.at[0], idxr.at[0], isems[p]).wait()
        pltpu.make_async_copy(row_hbm.at[0], idxc.at[0], isems[p]).wait()

    def _fire_gather(ztab, islot, dslot, p):
        pltpu.async_copy(ztab.at[idxr.at[islot]], zsrc.at[dslot], gsems[p])
        pltpu.async_copy(ztab.at[idxc.at[islot]], zdst.at[dslot], gsems[p])

    def _wait_gather(p):
        pltpu.make_async_copy(z0_hbm.at[pl.ds(0, CHUNK)], zsrc.at[0],
                              gsems[p]).wait()
        pltpu.make_async_copy(z0_hbm.at[pl.ds(0, CHUNK)], zdst.at[0],
                              gsems[p]).wait()

    def _fire_scatter(islot, dslot, p):
        pltpu.async_copy(wz.at[dslot], num_sh.at[idxr.at[islot]],
                         ssems[p], add=True)
        pltpu.async_copy(w2d.at[dslot], den_sh.at[idxr.at[islot]],
                         ssems[p], add=True)

    def _wait_scatter(p):
        pltpu.make_async_copy(z0_hbm.at[pl.ds(0, CHUNK)], wz.at[0],
                              ssems[p]).wait()
        pltpu.make_async_copy(den00.at[pl.ds(0, CHUNK)], w2d.at[0],
                              ssems[p]).wait()

    def _compute(d):
        def _grp(g, carry2):
            base = g * 16
            vecs = []
            bregs = []
            for e in range(16):
                a0 = zsrc[d, base + e, pl.ds(0, 16)]
                a1 = zsrc[d, base + e, pl.ds(16, 16)]
                b0 = zdst[d, base + e, pl.ds(0, 16)]
                b1 = zdst[d, base + e, pl.ds(16, 16)]
                bregs.append((b0, b1))
                vecs.append(a0 * b0 + a1 * b1)
            for r in range(4):
                half = len(vecs) // 2
                vecs = [_merge(vecs[i], vecs[i + half], r)
                        for i in range(half)]
            dots = vecs[0]
            sv = jnp.where(dots >= 0.0, dots, dots * 0.01)
            wv = jnp.exp(sv)
            w2d[d, pl.ds(g * 16, 16)] = wv
            for e in range(16):
                ws = _permute(wv, jnp.full((16,), e, jnp.int32))
                b0, b1 = bregs[e]
                wz[d, base + e, pl.ds(0, 16)] = b0 * ws
                wz[d, base + e, pl.ds(16, 16)] = b1 * ws
            return carry2
        lax.fori_loop(0, CHUNK // 16, _grp, 0)

    def _zv(i, _):
        zvec[pl.ds(i * 16, 16)] = jnp.zeros((16,), jnp.float32)
        return _
    lax.fori_loop(0, 448 // 16, _zv, 0)

    for ch in range(2):
        ztab = z0_hbm if ch == 0 else z1_hbm

        _fire_idx(0, 0, 0)
        _wait_idx(0)
        _fire_gather(ztab, 0, 0, 0)
        _fire_idx(1, 1, 1)

        def _zw(i, _):
            wz[0, i, pl.ds(0, 16)] = jnp.zeros((16,), jnp.float32)
            wz[0, i, pl.ds(16, 16)] = jnp.zeros((16,), jnp.float32)
            return _
        lax.fori_loop(0, CHUNK, _zw, 0)
        for j in range(24):
            pltpu.sync_copy(wz.at[0],
                            num_sh.at[pl.ds(tbase + j * 128, 128)])
        pltpu.sync_copy(wz.at[0, pl.ds(0, 64)],
                        num_sh.at[pl.ds(tbase + 3072, 64)])
        for j in range(7):
            pltpu.sync_copy(zvec, den_sh.at[pl.ds(tbase + j * 448, 448)])
        plsc.subcore_barrier()

        def _body(i, carry):
            for s in range(4):
                k = i * 4 + s
                p = s % 2
                if s < 2:
                    @pl.when(i > 0)
                    def _w1():
                        _wait_scatter(p)
                else:
                    _wait_scatter(p)
                if s < 2:
                    _fire_idx(k + 2, s + 2, p)
                else:
                    @pl.when(i < NITER - 1)
                    def _f2():
                        _fire_idx(k + 2, s - 2, p)
                if s < 3:
                    _wait_idx(1 - p)
                    _fire_gather(ztab, (s + 1) % 4, 1 - p, 1 - p)
                else:
                    @pl.when(i < NITER - 1)
                    def _f3():
                        _wait_idx(1 - p)
                        _fire_gather(ztab, 0, 1 - p, 1 - p)
                _wait_gather(p)
                _compute(p)
                _fire_scatter(s, p, p)
            return carry
        lax.fori_loop(0, NITER, _body, 0)
        _wait_scatter(0)
        _wait_scatter(1)
        plsc.subcore_barrier()

        pltpu.sync_copy(num_sh.at[pl.ds(tbase, ROWS_PER_TILE)],
                        num_out.at[ch, cid, pl.ds(tbase, ROWS_PER_TILE)])
        den_c0 = (den00, den10)[ch]
        den_c1 = (den01, den11)[ch]

        @pl.when(cid == 0)
        def _flush0():
            pltpu.sync_copy(den_sh.at[pl.ds(tbase, ROWS_PER_TILE)],
                            den_c0.at[pl.ds(tbase, ROWS_PER_TILE)])

        @pl.when(cid == 1)
        def _flush1():
            pltpu.sync_copy(den_sh.at[pl.ds(tbase, ROWS_PER_TILE)],
                            den_c1.at[pl.ds(tbase, ROWS_PER_TILE)])
        plsc.subcore_barrier()


def _sc_aggregate(z0, z1, row2d, col2d):
    mesh = plsc.VectorSubcoreMesh(core_axis_name="c", subcore_axis_name="s",
                                  num_cores=2, num_subcores=16)
    fn = pl.kernel(
        _sc_body,
        out_type=[
            jax.ShapeDtypeStruct((2, 2, N_PAD, CD), jnp.float32),
            jax.ShapeDtypeStruct((N_PAD,), jnp.float32),
            jax.ShapeDtypeStruct((N_PAD,), jnp.float32),
            jax.ShapeDtypeStruct((N_PAD,), jnp.float32),
            jax.ShapeDtypeStruct((N_PAD,), jnp.float32),
        ],
        mesh=mesh,
        compiler_params=pltpu.CompilerParams(needs_layout_passes=False,
                                             use_tc_tiling_on_sc=False),
        scratch_types=[
            pltpu.VMEM((4, CHUNK), jnp.int32),
            pltpu.VMEM((4, CHUNK), jnp.int32),
            pltpu.VMEM((2, CHUNK, CD), jnp.float32),
            pltpu.VMEM((2, CHUNK, CD), jnp.float32),
            pltpu.VMEM((2, CHUNK, CD), jnp.float32),
            pltpu.VMEM((2, CHUNK), jnp.float32),
            pltpu.VMEM((448,), jnp.float32),
            pltpu.VMEM_SHARED((N_PAD, CD), jnp.float32),
            pltpu.VMEM_SHARED((N_PAD,), jnp.float32),
            pltpu.SemaphoreType.DMA,
            pltpu.SemaphoreType.DMA,
            pltpu.SemaphoreType.DMA,
            pltpu.SemaphoreType.DMA,
            pltpu.SemaphoreType.DMA,
            pltpu.SemaphoreType.DMA,
        ],
    )
    return fn(z0, z1, row2d, col2d)


NODES_PER_TILE_F = N_PAD // 32
FCHUNK = 224


def _fin_body(num, d00, d01, d10, d11, emb_tab, out128,
              nb00, nb01, nb10, nb11, db00, db01, db10, db11, eb, ob):
    cid = lax.axis_index("c")
    sid = lax.axis_index("s")
    wid = sid * 2 + cid

    def _chunk(c, carry):
        nb = wid * NODES_PER_TILE_F + c * FCHUNK
        for core, (nref0, nref1) in ((0, (nb00, nb10)), (1, (nb01, nb11))):
            pltpu.sync_copy(num.at[0, core, pl.ds(nb, FCHUNK)], nref0)
            pltpu.sync_copy(num.at[1, core, pl.ds(nb, FCHUNK)], nref1)
        for src, dst in ((d00, db00), (d01, db01), (d10, db10), (d11, db11)):
            pltpu.sync_copy(src.at[pl.ds(nb, FCHUNK)], dst)
        pltpu.sync_copy(emb_tab.at[pl.ds(nb, FCHUNK)], eb)

        def _grp(g, carry2):
            g16 = g * 16
            dv0 = db00[pl.ds(g16, 16)] + db01[pl.ds(g16, 16)]
            rec0 = 1.0 / (dv0 + 1e-16)
            dv1 = db10[pl.ds(g16, 16)] + db11[pl.ds(g16, 16)]
            rec1 = 1.0 / (dv1 + 1e-16)
            for e in range(16):
                n = g16 + e
                r0e = _permute(rec0, jnp.full((16,), e, jnp.int32))
                r1e = _permute(rec1, jnp.full((16,), e, jnp.int32))
                a0 = (nb00[n, pl.ds(0, 16)] + nb01[n, pl.ds(0, 16)]) * r0e
                a1 = (nb00[n, pl.ds(16, 16)] + nb01[n, pl.ds(16, 16)]) * r0e
                b0 = (nb10[n, pl.ds(0, 16)] + nb11[n, pl.ds(0, 16)]) * r1e
                b1 = (nb10[n, pl.ds(16, 16)] + nb11[n, pl.ds(16, 16)]) * r1e
                er = g * 8 + e // 2
                ec = (e % 2) * 64
                ob[er, pl.ds(ec, 16)] = 0.5 * (eb[n, pl.ds(0, 16)] + a0)
                ob[er, pl.ds(ec + 16, 16)] = 0.5 * (eb[n, pl.ds(16, 16)] + a1)
                ob[er, pl.ds(ec + 32, 16)] = 0.5 * (eb[n, pl.ds(32, 16)] + b0)
                ob[er, pl.ds(ec + 48, 16)] = 0.5 * (eb[n, pl.ds(48, 16)] + b1)
            return carry2
        lax.fori_loop(0, FCHUNK // 16, _grp, 0)
        pltpu.sync_copy(ob, out128.at[pl.ds(nb // 2, FCHUNK // 2)])
        return carry
    lax.fori_loop(0, NODES_PER_TILE_F // FCHUNK, _chunk, 0)


def _sc_finalize(num, d00, d01, d10, d11, emb_tab):
    mesh = plsc.VectorSubcoreMesh(core_axis_name="c", subcore_axis_name="s",
                                  num_cores=2, num_subcores=16)
    fn = pl.kernel(
        _fin_body,
        out_type=jax.ShapeDtypeStruct((N_PAD * D // 128, 128), jnp.float32),
        mesh=mesh,
        compiler_params=pltpu.CompilerParams(needs_layout_passes=False,
                                             use_tc_tiling_on_sc=False),
        scratch_types=[
            pltpu.VMEM((FCHUNK, CD), jnp.float32),
            pltpu.VMEM((FCHUNK, CD), jnp.float32),
            pltpu.VMEM((FCHUNK, CD), jnp.float32),
            pltpu.VMEM((FCHUNK, CD), jnp.float32),
            pltpu.VMEM((FCHUNK,), jnp.float32),
            pltpu.VMEM((FCHUNK,), jnp.float32),
            pltpu.VMEM((FCHUNK,), jnp.float32),
            pltpu.VMEM((FCHUNK,), jnp.float32),
            pltpu.VMEM((FCHUNK, D), jnp.float32),
            pltpu.VMEM((FCHUNK // 2, 128), jnp.float32),
        ],
    )
    return fn(num, d00, d01, d10, d11, emb_tab)


def kernel(user_emb, item_emb, user_preference_sample, edge_index,
           W_ch, b_ch, W_merge, b_merge):
    u = _merge_users(user_preference_sample, user_emb, W_merge, b_merge)
    emb = jnp.concatenate([u, item_emb], axis=0)
    embT = jnp.pad(emb, ((0, N_PAD - N_NODE), (0, 0))).T

    z0T, z1T = _channel_proj(embT, W_ch.transpose(0, 2, 1),
                             b_ch.transpose(0, 2, 1))
    z0t, z1t, emb_tab = _sc_repack(z0T, z1T, embT)

    pad = N_NODE + (jnp.arange(E_PAD - N_EDGE, dtype=jnp.int32)
                    % (N_PAD - N_NODE))
    row2d = jnp.concatenate([edge_index[0], pad]).reshape(E_PAD // 128, 128)
    col2d = jnp.concatenate([edge_index[1], pad]).reshape(E_PAD // 128, 128)

    num, d00, d01, d10, d11 = _sc_aggregate(z0t, z1t, row2d, col2d)

    out128 = _sc_finalize(num, d00, d01, d10, d11, emb_tab)
    return out128.reshape(N_PAD, D)[:N_NODE]
